# baseline (speedup 1.0000x reference)
import functools

import jax
import jax.numpy as jnp
from jax import lax
from jax.experimental import pallas as pl
from jax.experimental.pallas import tpu as pltpu
from jax.experimental.pallas import tpu_sc as plsc

N_CTRS = 1000
D_IN = 64
D_OUT = 64
N_SMPS = 4096
ROW_T = 16
ROW_L = 128


_SBLK = 256
_NSB = N_SMPS // _SBLK


def _top2_body(xsq_ref, csq_ref, x_ref, ctrs_ref, idx_ref):
    xb = x_ref[...]
    cb = ctrs_ref[...]
    m = lax.dot_general(xb, cb, (((1,), (1,)), ((), ())),
                        preferred_element_type=jnp.float32)
    d2 = (xsq_ref[...] - 2.0 * m) + csq_ref[...]
    iota = lax.broadcasted_iota(jnp.int32, d2.shape, 1)
    m1 = jnp.min(d2, axis=1, keepdims=True)
    i1 = jnp.min(jnp.where(d2 == m1, iota, jnp.int32(1 << 30)),
                 axis=1, keepdims=True)
    d2b = jnp.where(iota == i1, jnp.float32(3e38), d2)
    m2 = jnp.min(d2b, axis=1, keepdims=True)
    i2 = jnp.min(jnp.where(d2b == m2, iota, jnp.int32(1 << 30)),
                 axis=1, keepdims=True)
    pad = jnp.zeros((d2.shape[0], 6), jnp.int32)
    idx_ref[...] = jnp.concatenate([i1, i2, pad], axis=1)


def _top2(x, ctrs, x_sq, c_sq):
    return pl.pallas_call(
        _top2_body,
        grid=(_NSB,),
        in_specs=[
            pl.BlockSpec((_SBLK, 1), lambda i: (i, 0)),
            pl.BlockSpec((1, N_CTRS), lambda i: (0, 0)),
            pl.BlockSpec((_SBLK, D_IN), lambda i: (i, 0)),
            pl.BlockSpec((N_CTRS, D_IN), lambda i: (0, 0)),
        ],
        out_specs=pl.BlockSpec((_SBLK, 8), lambda i: (i, 0)),
        out_shape=jax.ShapeDtypeStruct((N_SMPS, 8), jnp.int32),
    )(x_sq, c_sq, x, ctrs)


_CBLK = 200
_NCB = N_CTRS // _CBLK


def _rep_planes(v_ref, sel_ref):
    hp = lax.Precision.HIGHEST
    vs = lax.dot_general(v_ref[...], sel_ref[...], (((1,), (0,)), ((), ())),
                         precision=hp, preferred_element_type=jnp.float32)
    vel = vs[:, 0 * ROW_T:1 * ROW_T]
    vol = vs[:, 1 * ROW_T:2 * ROW_T]
    veh = vs[:, 2 * ROW_T:3 * ROW_T]
    voh = vs[:, 3 * ROW_T:4 * ROW_T]
    lane = lax.broadcasted_iota(jnp.int32, (1, 1, ROW_L), 2)
    rep_l = jnp.where(lane < D_OUT, vel[:, :, None], vol[:, :, None])
    rep_h = jnp.where(lane < D_OUT, veh[:, :, None], voh[:, :, None])
    return rep_l, rep_h


def _unpack(w_ref):
    u = lax.bitcast_convert_type(w_ref[...], jnp.int32)
    lo = lax.bitcast_convert_type(
        lax.shift_left(u, jnp.int32(16)), jnp.float32)
    hi = lax.bitcast_convert_type(
        lax.bitwise_and(u, jnp.int32(-65536)), jnp.float32)
    return lo, hi


def _bias_body(ctrs_ref, sel_ref, wpk_ref, off_ref, bias_ref):
    lo, hi = _unpack(wpk_ref)
    crep_l, crep_h = _rep_planes(ctrs_ref, sel_ref)
    q = jnp.sum(lo * crep_l + hi * crep_h, axis=1)
    cw = q[:, :D_OUT] + q[:, D_OUT:]
    b = off_ref[...] - cw
    bias_ref[...] = jnp.concatenate(
        [b, jnp.zeros((b.shape[0], 128 - D_OUT), jnp.float32)], axis=1)


def _bias_table(ctrs, sel, wtab, offsets):
    return pl.pallas_call(
        _bias_body,
        grid=(_NCB,),
        in_specs=[
            pl.BlockSpec((_CBLK, D_IN), lambda i: (i, 0)),
            pl.BlockSpec((D_IN, 4 * ROW_T), lambda i: (0, 0)),
            pl.BlockSpec((_CBLK, ROW_T, ROW_L), lambda i: (i, 0, 0)),
            pl.BlockSpec((_CBLK, D_OUT), lambda i: (i, 0)),
        ],
        out_specs=pl.BlockSpec((_CBLK, 128), lambda i: (i, 0)),
        out_shape=jax.ShapeDtypeStruct((N_CTRS, 128), jnp.float32),
    )(ctrs, sel, wtab, offsets)


_NW = 32
_NPAIR = 2 * N_SMPS
_BPW = _NPAIR // _NW
_G = 16
_NCH = _BPW // _G


def _sc_gather_w_build(npair):
    bpw = npair // _NW
    nch = bpw // _G
    mesh = plsc.VectorSubcoreMesh(core_axis_name="c", subcore_axis_name="s")

    @functools.partial(
        pl.kernel,
        mesh=mesh,
        out_type=jax.ShapeDtypeStruct((npair, ROW_T, ROW_L), jnp.float32),
        scratch_types=[
            pltpu.VMEM((bpw,), jnp.int32),
            pltpu.VMEM((_G, ROW_T, ROW_L), jnp.float32),
            pltpu.VMEM((_G, ROW_T, ROW_L), jnp.float32),
            pltpu.SemaphoreType.DMA,
            pltpu.SemaphoreType.DMA,
            pltpu.SemaphoreType.DMA,
            pltpu.SemaphoreType.DMA,
        ],
    )
    def sc_gather_w(wtab, idx, wsel, idx_v, rows0, rows1, sg0, sg1, so0, so1):
        wid = lax.axis_index("s") * 2 + lax.axis_index("c")
        base = wid * bpw
        pltpu.sync_copy(idx.at[pl.ds(base, bpw)], idx_v)

        rows = (rows0, rows1)
        sg = (sg0, sg1)
        so = (so0, so1)

        def start_gather(c, b):
            return pltpu.async_copy(
                wtab.at[idx_v.at[pl.ds(c * _G, _G)]], rows[b], sg[b])

        outs = [None, None]
        g = [None, None]
        g[0] = start_gather(0, 0)
        for c in range(nch):
            b = c & 1
            ob = b ^ 1
            g[b].wait()
            if c + 1 < nch:
                if outs[ob] is not None:
                    outs[ob].wait()
                g[ob] = start_gather(c + 1, ob)
            outs[b] = pltpu.async_copy(
                rows[b], wsel.at[pl.ds(base + c * _G, _G)], so[b])
        outs[0].wait()
        outs[1].wait()

    return sc_gather_w


_sc_gather_w = _sc_gather_w_build(_NPAIR)


_KBLK = 512
_NKB = N_SMPS // _KBLK


def _combine_body(x_ref, sel_ref, idx_ref, bias_ref, wa_ref, wb_ref, yt_ref):
    xrep_l, xrep_h = _rep_planes(x_ref, sel_ref)
    lo_a, hi_a = _unpack(wa_ref)
    lo_b, hi_b = _unpack(wb_ref)
    q = jnp.sum((lo_a + lo_b) * xrep_l + (hi_a + hi_b) * xrep_h, axis=1)
    y = q[:, :D_OUT] + q[:, D_OUT:]
    ids = idx_ref[...]
    iota = lax.broadcasted_iota(jnp.int32, (ids.shape[0], N_CTRS), 1)
    m = ((iota == ids[:, 0:1]).astype(jnp.float32)
         + (iota == ids[:, 1:2]).astype(jnp.float32))
    yb = lax.dot_general(m, bias_ref[...], (((1,), (0,)), ((), ())),
                         precision=lax.Precision.HIGHEST,
                         preferred_element_type=jnp.float32)
    y = y + yb[:, :D_OUT]
    yt_ref[...] = y.T


def _combine(x, sel, idx8, bias, wsel):
    hb = N_SMPS // _KBLK
    return pl.pallas_call(
        _combine_body,
        grid=(_NKB,),
        in_specs=[
            pl.BlockSpec((_KBLK, D_IN), lambda i: (i, 0)),
            pl.BlockSpec((D_IN, 4 * ROW_T), lambda i: (0, 0)),
            pl.BlockSpec((_KBLK, 8), lambda i: (i, 0)),
            pl.BlockSpec((N_CTRS, 128), lambda i: (0, 0)),
            pl.BlockSpec((_KBLK, ROW_T, ROW_L), lambda i: (i, 0, 0)),
            pl.BlockSpec((_KBLK, ROW_T, ROW_L), lambda i: (i + hb, 0, 0)),
        ],
        out_specs=pl.BlockSpec((D_OUT, _KBLK), lambda i: (0, i)),
        out_shape=jax.ShapeDtypeStruct((D_OUT, N_SMPS), jnp.float32),
    )(x, sel, idx8, bias, wsel, wsel)


def kernel(x, ctrs, wts, offsets):
    x_sq = jnp.sum(x * x, axis=1, keepdims=True)
    c_sq = jnp.sum(ctrs * ctrs, axis=1)[None, :]
    idx8 = _top2(x, ctrs, x_sq, c_sq)
    idx_flat = jnp.concatenate([idx8[:, 0], idx8[:, 1]])

    u = lax.bitcast_convert_type(wts, jnp.int32)
    r = u + jnp.int32(0x7FFF) + lax.bitwise_and(
        lax.shift_right_arithmetic(u, jnp.int32(16)), jnp.int32(1))
    r5 = r.reshape(N_CTRS, 2, ROW_T, 2, D_OUT)
    pk = lax.bitwise_or(
        lax.shift_right_logical(r5[:, 0], jnp.int32(16)),
        lax.bitwise_and(r5[:, 1], jnp.int32(-65536)))
    wtab = lax.bitcast_convert_type(
        pk.reshape(N_CTRS, ROW_T, ROW_L), jnp.float32)

    eye = jnp.eye(D_IN, dtype=jnp.float32)
    sel = jnp.concatenate(
        [eye[:, 0:2 * ROW_T:2], eye[:, 1:2 * ROW_T:2],
         eye[:, 2 * ROW_T::2], eye[:, 2 * ROW_T + 1::2]], axis=1)

    wsel = _sc_gather_w(wtab, idx_flat)
    bias = _bias_table(ctrs, sel, wtab, offsets)
    return _combine(x, sel, idx8, bias, wsel).T


# speedup vs baseline: 1.0193x; 1.0193x over previous
import functools

import jax
import jax.numpy as jnp
from jax import lax
from jax.experimental import pallas as pl
from jax.experimental.pallas import tpu as pltpu
from jax.experimental.pallas import tpu_sc as plsc

N_CTRS = 1000
D_IN = 64
D_OUT = 64
N_SMPS = 4096
ROW_T = 16
ROW_L = 128


_SBLK = 256
_NSB = N_SMPS // _SBLK


def _top2_body(xsq_ref, csq_ref, x_ref, ctrs_ref, idx_ref):
    xb = x_ref[...]
    cb = ctrs_ref[...]
    m = lax.dot_general(xb, cb, (((1,), (1,)), ((), ())),
                        preferred_element_type=jnp.float32)
    d2 = (xsq_ref[...] - 2.0 * m) + csq_ref[...]
    iota = lax.broadcasted_iota(jnp.int32, d2.shape, 1)
    m1 = jnp.min(d2, axis=1, keepdims=True)
    i1 = jnp.min(jnp.where(d2 == m1, iota, jnp.int32(1 << 30)),
                 axis=1, keepdims=True)
    d2b = jnp.where(iota == i1, jnp.float32(3e38), d2)
    m2 = jnp.min(d2b, axis=1, keepdims=True)
    i2 = jnp.min(jnp.where(d2b == m2, iota, jnp.int32(1 << 30)),
                 axis=1, keepdims=True)
    pad = jnp.zeros((d2.shape[0], 6), jnp.int32)
    idx_ref[...] = jnp.concatenate([i1, i2, pad], axis=1)


def _top2(x, ctrs, x_sq, c_sq):
    return pl.pallas_call(
        _top2_body,
        grid=(_NSB,),
        in_specs=[
            pl.BlockSpec((_SBLK, 1), lambda i: (i, 0)),
            pl.BlockSpec((1, N_CTRS), lambda i: (0, 0)),
            pl.BlockSpec((_SBLK, D_IN), lambda i: (i, 0)),
            pl.BlockSpec((N_CTRS, D_IN), lambda i: (0, 0)),
        ],
        out_specs=pl.BlockSpec((_SBLK, 8), lambda i: (i, 0)),
        out_shape=jax.ShapeDtypeStruct((N_SMPS, 8), jnp.int32),
    )(x_sq, c_sq, x, ctrs)


_CBLK = 200
_NCB = N_CTRS // _CBLK


def _rep_planes(v_ref, sel_ref):
    hp = lax.Precision.HIGHEST
    vs = lax.dot_general(v_ref[...], sel_ref[...], (((1,), (0,)), ((), ())),
                         precision=hp, preferred_element_type=jnp.float32)
    vel = vs[:, 0 * ROW_T:1 * ROW_T]
    vol = vs[:, 1 * ROW_T:2 * ROW_T]
    veh = vs[:, 2 * ROW_T:3 * ROW_T]
    voh = vs[:, 3 * ROW_T:4 * ROW_T]
    lane = lax.broadcasted_iota(jnp.int32, (1, 1, ROW_L), 2)
    rep_l = jnp.where(lane < D_OUT, vel[:, :, None], vol[:, :, None])
    rep_h = jnp.where(lane < D_OUT, veh[:, :, None], voh[:, :, None])
    return rep_l, rep_h


def _unpack(w_ref):
    u = lax.bitcast_convert_type(w_ref[...], jnp.int32)
    lo = lax.bitcast_convert_type(
        lax.shift_left(u, jnp.int32(16)), jnp.float32)
    hi = lax.bitcast_convert_type(
        lax.bitwise_and(u, jnp.int32(-65536)), jnp.float32)
    return lo, hi


def _bias_body(ctrs_ref, sel_ref, wpk_ref, off_ref, bias_ref):
    lo, hi = _unpack(wpk_ref)
    crep_l, crep_h = _rep_planes(ctrs_ref, sel_ref)
    q = jnp.sum(lo * crep_l + hi * crep_h, axis=1)
    cw = q[:, :D_OUT] + q[:, D_OUT:]
    b = off_ref[...] - cw
    bias_ref[...] = jnp.concatenate(
        [b, jnp.zeros((b.shape[0], 128 - D_OUT), jnp.float32)], axis=1)


def _bias_table(ctrs, sel, wtab, offsets):
    return pl.pallas_call(
        _bias_body,
        grid=(_NCB,),
        in_specs=[
            pl.BlockSpec((_CBLK, D_IN), lambda i: (i, 0)),
            pl.BlockSpec((D_IN, 4 * ROW_T), lambda i: (0, 0)),
            pl.BlockSpec((_CBLK, ROW_T, ROW_L), lambda i: (i, 0, 0)),
            pl.BlockSpec((_CBLK, D_OUT), lambda i: (i, 0)),
        ],
        out_specs=pl.BlockSpec((_CBLK, 128), lambda i: (i, 0)),
        out_shape=jax.ShapeDtypeStruct((N_CTRS, 128), jnp.float32),
    )(ctrs, sel, wtab, offsets)


_NW = 32
_NPAIR = 2 * N_SMPS
_BPW = _NPAIR // _NW
_G = 16
_NCH = _BPW // _G


def _sc_gather_w_build(npair):
    bpw = npair // _NW
    nch = bpw // _G
    mesh = plsc.VectorSubcoreMesh(core_axis_name="c", subcore_axis_name="s")

    @functools.partial(
        pl.kernel,
        mesh=mesh,
        out_type=jax.ShapeDtypeStruct((npair, ROW_T, ROW_L), jnp.float32),
        scratch_types=[
            pltpu.VMEM((bpw,), jnp.int32),
            pltpu.VMEM((_G, ROW_T, ROW_L), jnp.float32),
            pltpu.VMEM((_G, ROW_T, ROW_L), jnp.float32),
            pltpu.SemaphoreType.DMA,
            pltpu.SemaphoreType.DMA,
            pltpu.SemaphoreType.DMA,
            pltpu.SemaphoreType.DMA,
        ],
    )
    def sc_gather_w(wtab, idx, wsel, idx_v, rows0, rows1, sg0, sg1, so0, so1):
        wid = lax.axis_index("s") * 2 + lax.axis_index("c")
        base = wid * bpw
        pltpu.sync_copy(idx.at[pl.ds(base, bpw)], idx_v)

        rows = (rows0, rows1)
        sg = (sg0, sg1)
        so = (so0, so1)

        def start_gather(c, b):
            return pltpu.async_copy(
                wtab.at[idx_v.at[pl.ds(c * _G, _G)]], rows[b], sg[b])

        outs = [None, None]
        g = [None, None]
        g[0] = start_gather(0, 0)
        for c in range(nch):
            b = c & 1
            ob = b ^ 1
            g[b].wait()
            if c + 1 < nch:
                if outs[ob] is not None:
                    outs[ob].wait()
                g[ob] = start_gather(c + 1, ob)
            outs[b] = pltpu.async_copy(
                rows[b], wsel.at[pl.ds(base + c * _G, _G)], so[b])
        outs[0].wait()
        outs[1].wait()

    return sc_gather_w


def _sc_gather_b_build():
    mesh = plsc.VectorSubcoreMesh(core_axis_name="c", subcore_axis_name="s")

    @functools.partial(
        pl.kernel,
        mesh=mesh,
        out_type=jax.ShapeDtypeStruct((_NPAIR, 128), jnp.float32),
        scratch_types=[
            pltpu.VMEM((_BPW,), jnp.int32),
            pltpu.VMEM((_BPW, 128), jnp.float32),
            pltpu.SemaphoreType.DMA,
        ],
    )
    def sc_gather_b(bias, idx, bsel, idx_v, brows_v, sg0):
        wid = lax.axis_index("s") * 2 + lax.axis_index("c")
        base = wid * _BPW
        pltpu.sync_copy(idx.at[pl.ds(base, _BPW)], idx_v)
        pltpu.async_copy(bias.at[idx_v], brows_v, sg0).wait()
        pltpu.sync_copy(brows_v, bsel.at[pl.ds(base, _BPW)])

    return sc_gather_b


_sc_gather_w = _sc_gather_w_build(_NPAIR)
_sc_gather_b = _sc_gather_b_build()


_KBLK = 512
_NKB = N_SMPS // _KBLK


def _combine_body(x_ref, sel_ref, wa_ref, wb_ref, ba_ref, bb_ref, yt_ref):
    xrep_l, xrep_h = _rep_planes(x_ref, sel_ref)
    lo_a, hi_a = _unpack(wa_ref)
    lo_b, hi_b = _unpack(wb_ref)
    q = jnp.sum((lo_a + lo_b) * xrep_l + (hi_a + hi_b) * xrep_h, axis=1)
    y = q[:, :D_OUT] + q[:, D_OUT:]
    y = y + ba_ref[...][:, :D_OUT] + bb_ref[...][:, :D_OUT]
    yt_ref[...] = y.T


def _combine(x, sel, wsel, bsel):
    hb = N_SMPS // _KBLK
    return pl.pallas_call(
        _combine_body,
        grid=(_NKB,),
        in_specs=[
            pl.BlockSpec((_KBLK, D_IN), lambda i: (i, 0)),
            pl.BlockSpec((D_IN, 4 * ROW_T), lambda i: (0, 0)),
            pl.BlockSpec((_KBLK, ROW_T, ROW_L), lambda i: (i, 0, 0)),
            pl.BlockSpec((_KBLK, ROW_T, ROW_L), lambda i: (i + hb, 0, 0)),
            pl.BlockSpec((_KBLK, 128), lambda i: (i, 0)),
            pl.BlockSpec((_KBLK, 128), lambda i: (i + hb, 0)),
        ],
        out_specs=pl.BlockSpec((D_OUT, _KBLK), lambda i: (0, i)),
        out_shape=jax.ShapeDtypeStruct((D_OUT, N_SMPS), jnp.float32),
    )(x, sel, wsel, wsel, bsel, bsel)


def kernel(x, ctrs, wts, offsets):
    x_sq = jnp.sum(x * x, axis=1, keepdims=True)
    c_sq = jnp.sum(ctrs * ctrs, axis=1)[None, :]
    idx8 = _top2(x, ctrs, x_sq, c_sq)
    idx_flat = jnp.concatenate([idx8[:, 0], idx8[:, 1]])

    u = lax.bitcast_convert_type(wts, jnp.int32)
    r = u + jnp.int32(0x7FFF) + lax.bitwise_and(
        lax.shift_right_arithmetic(u, jnp.int32(16)), jnp.int32(1))
    r5 = r.reshape(N_CTRS, 2, ROW_T, 2, D_OUT)
    pk = lax.bitwise_or(
        lax.shift_right_logical(r5[:, 0], jnp.int32(16)),
        lax.bitwise_and(r5[:, 1], jnp.int32(-65536)))
    wtab = lax.bitcast_convert_type(
        pk.reshape(N_CTRS, ROW_T, ROW_L), jnp.float32)

    eye = jnp.eye(D_IN, dtype=jnp.float32)
    sel = jnp.concatenate(
        [eye[:, 0:2 * ROW_T:2], eye[:, 1:2 * ROW_T:2],
         eye[:, 2 * ROW_T::2], eye[:, 2 * ROW_T + 1::2]], axis=1)

    wsel = _sc_gather_w(wtab, idx_flat)
    bias = _bias_table(ctrs, sel, wtab, offsets)
    bsel = _sc_gather_b(bias, idx_flat)
    return _combine(x, sel, wsel, bsel).T
